# Initial kernel scaffold; baseline (speedup 1.0000x reference)
import jax
import jax.numpy as jnp
from jax.experimental import pallas as pl


def kernel(x, src, tgt, Mtgt, W0, b0, W1, b1, W2, b2, W3, b3, g1w, g1b, g2w, g2b):
    raise NotImplementedError("write your pallas kernel here")


# speedup vs baseline: 4.4644x; 4.4644x over previous
import functools

import jax
import jax.numpy as jnp
from jax import lax
from jax.experimental import pallas as pl
from jax.experimental.pallas import tpu as pltpu
from jax.experimental.pallas import tpu_sc as plsc

N = 10000
NPAD = 10240
E = 320000
G = 80
N_TILES = 32
EPT = E // N_TILES
WPT = EPT // G
RPT = NPAD // 16
ZCH = 128
EPS = 1e-5


def _make_sc_agg(F):
    mesh = plsc.VectorSubcoreMesh(core_axis_name="c", subcore_axis_name="s")

    @functools.partial(
        pl.kernel,
        out_type=jax.ShapeDtypeStruct((2, NPAD, F), jnp.float32),
        mesh=mesh,
        scratch_types=[
            pltpu.VMEM((G,), jnp.int32),
            pltpu.VMEM((G,), jnp.int32),
            pltpu.VMEM((G, F), jnp.float32),
            pltpu.VMEM((ZCH, F), jnp.float32),
            pltpu.VMEM_SHARED((NPAD, F), jnp.float32),
        ],
    )
    def agg(h_hbm, src_hbm, tgt_hbm, zeros_hbm, out_hbm,
            src_v, tgt_v, rows_v, stage_v, acc_sh):
        c = lax.axis_index("c")
        s = lax.axis_index("s")
        wid = c * 16 + s

        pltpu.sync_copy(zeros_hbm, stage_v)
        for k in range(RPT // ZCH):
            pltpu.sync_copy(stage_v, acc_sh.at[pl.ds(s * RPT + k * ZCH, ZCH)])
        plsc.subcore_barrier()

        ebase = wid * EPT

        @pl.loop(0, WPT)
        def _(w):
            pltpu.sync_copy(src_hbm.at[pl.ds(ebase + w * G, G)], src_v)
            pltpu.sync_copy(tgt_hbm.at[pl.ds(ebase + w * G, G)], tgt_v)
            pltpu.sync_copy(h_hbm.at[src_v], rows_v)
            pltpu.sync_copy(rows_v, acc_sh.at[tgt_v], add=True)

        plsc.subcore_barrier()

        for k in range(RPT // ZCH):
            start = s * RPT + k * ZCH
            pltpu.sync_copy(acc_sh.at[pl.ds(start, ZCH)], stage_v)
            pltpu.sync_copy(stage_v, out_hbm.at[c].at[pl.ds(start, ZCH)])

    return agg


_sc_agg_128 = _make_sc_agg(128)


BR = 1000


def _row_spec(shape_f):
    return pl.BlockSpec((BR,) + shape_f, lambda i: (i,) + (0,) * len(shape_f))


def _full_spec(shape):
    return pl.BlockSpec(shape, lambda i: (0,) * len(shape))


def _stage_a_body(x_ref, w_ref, b_ref, t_ref):
    t_ref[...] = jnp.dot(x_ref[...], w_ref[...],
                         preferred_element_type=jnp.float32) + b_ref[...]


def _stage_a(x, W, b):
    F = W.shape[1]
    return pl.pallas_call(
        _stage_a_body,
        grid=(N // BR,),
        in_specs=[_row_spec((128,)), _full_spec((128, F)), _full_spec((1, F))],
        out_specs=_row_spec((F,)),
        out_shape=jax.ShapeDtypeStruct((N, F), jnp.float32),
    )(x, W, b.reshape(1, F))


def _stage_b_body(p_ref, m_ref, w_ref, b_ref, h_ref, t_ref):
    h = jax.nn.relu(m_ref[...] * (p_ref[0] + p_ref[1]))
    h_ref[...] = h
    t_ref[...] = jnp.dot(h, w_ref[...],
                         preferred_element_type=jnp.float32) + b_ref[...]


def _stage_b(p, Mtgt, W, b):
    F = W.shape[1]
    return pl.pallas_call(
        _stage_b_body,
        grid=(N // BR,),
        in_specs=[
            pl.BlockSpec((2, BR, 128), lambda i: (0, i, 0)),
            _row_spec((1,)),
            _full_spec((128, F)),
            _full_spec((1, F)),
        ],
        out_specs=[_row_spec((128,)), _row_spec((F,))],
        out_shape=[
            jax.ShapeDtypeStruct((N, 128), jnp.float32),
            jax.ShapeDtypeStruct((N, F), jnp.float32),
        ],
    )(p, Mtgt, W, b.reshape(1, F))


def _stage_mid_body(p_ref, m_ref, g_ref, gw_ref, gb_ref, r_ref, w_ref, b_ref,
                    h_ref, t_ref):
    z = jax.nn.relu(m_ref[...] * (p_ref[0] + p_ref[1]))
    mean = jnp.dot(z, g_ref[...], preferred_element_type=jnp.float32)
    d = z - mean
    var = jnp.dot(d * d, g_ref[...], preferred_element_type=jnp.float32)
    gn = d * lax.rsqrt(var + EPS) * gw_ref[...] + gb_ref[...]
    h = gn + r_ref[...]
    h_ref[...] = h
    t_ref[...] = jnp.dot(h, w_ref[...],
                         preferred_element_type=jnp.float32) + b_ref[...]


def _stage_mid(p, Mtgt, Gmat, gw, gb, resid, W, b):
    F = W.shape[1]
    return pl.pallas_call(
        _stage_mid_body,
        grid=(N // BR,),
        in_specs=[
            pl.BlockSpec((2, BR, 128), lambda i: (0, i, 0)),
            _row_spec((1,)),
            _full_spec((128, 128)),
            _full_spec((1, 128)),
            _full_spec((1, 128)),
            _row_spec((128,)),
            _full_spec((128, F)),
            _full_spec((1, F)),
        ],
        out_specs=[_row_spec((128,)), _row_spec((F,))],
        out_shape=[
            jax.ShapeDtypeStruct((N, 128), jnp.float32),
            jax.ShapeDtypeStruct((N, F), jnp.float32),
        ],
    )(p, Mtgt, Gmat, gw.reshape(1, 128), gb.reshape(1, 128), resid, W,
      b.reshape(1, F))


def _stage_e_body(p_ref, m_ref, o_ref):
    C = o_ref.shape[1]
    o = (m_ref[...] * (p_ref[0] + p_ref[1]))[:, :C]
    mx = jnp.max(o, axis=1, keepdims=True)
    lse = jnp.log(jnp.sum(jnp.exp(o - mx), axis=1, keepdims=True)) + mx
    o_ref[...] = o - lse


def _stage_e(p, Mtgt, C):
    return pl.pallas_call(
        _stage_e_body,
        grid=(N // BR,),
        in_specs=[
            pl.BlockSpec((2, BR, 128), lambda i: (0, i, 0)),
            _row_spec((1,)),
        ],
        out_specs=_row_spec((C,)),
        out_shape=jax.ShapeDtypeStruct((N, C), jnp.float32),
    )(p, Mtgt)


def kernel(x, src, tgt, Mtgt, W0, b0, W1, b1, W2, b2, W3, b3,
           g1w, g1b, g2w, g2b):
    zeros = jnp.zeros((ZCH, 128), jnp.float32)
    Gmat = jnp.kron(jnp.eye(32, dtype=jnp.float32),
                    jnp.full((4, 4), 0.25, jnp.float32))
    nclass = W3.shape[1]
    W3p = jnp.pad(W3, ((0, 0), (0, 128 - nclass)))
    b3p = jnp.pad(b3, (0, 128 - nclass))

    t0 = _stage_a(x, W0, b0)
    p0 = _sc_agg_128(t0, src, tgt, zeros)
    h0, t1 = _stage_b(p0, Mtgt, W1, b1)
    p1 = _sc_agg_128(t1, src, tgt, zeros)
    h1, t2 = _stage_mid(p1, Mtgt, Gmat, g1w, g1b, h0, W2, b2)
    p2 = _sc_agg_128(t2, src, tgt, zeros)
    _, t3 = _stage_mid(p2, Mtgt, Gmat, g2w, g2b, h1, W3p, b3p)
    p3 = _sc_agg_128(t3, src, tgt, zeros)
    return _stage_e(p3, Mtgt, nclass)
